# baseline (speedup 1.0000x reference)
import functools
import math

import jax
import jax.numpy as jnp
import numpy as np
from jax import lax
from jax.experimental import pallas as pl
from jax.experimental.pallas import tpu as pltpu
from jax.experimental.pallas import tpu_sc as plsc

S = 2048
H = 1024
NH = 16
NKV = 8
HD = 64
I = 3584
E = 8
K = 2
THETA = 10000.0
EPS = 1e-06
NEG = float(jnp.finfo(jnp.float32).min)

_BS1 = 256
_BQ = 256
_BS3 = 512
_BI = 256
_GBS = 128
_PMAX = S * K + E * _GBS
_NB = _PMAX // _GBS
_NW = 32


def _dot_t(a, b):
    return lax.dot_general(a, b, (((1,), (1,)), ((), ())),
                           preferred_element_type=jnp.float32)


def _qkv_body(h_ref, ln1_ref, cq_ref, sq_ref, ck_ref, sk_ref,
              wq_ref, wk_ref, wv_ref, q_ref, k_ref, v_ref):
    x = h_ref[...]
    var = jnp.mean(x * x, axis=1, keepdims=True)
    xn = (x * lax.rsqrt(var + EPS)) * ln1_ref[...]
    q = _dot_t(xn, wq_ref[...])
    k = _dot_t(xn, wk_ref[...])
    v = _dot_t(xn, wv_ref[...])
    hq = NH * 32
    qrot = jnp.concatenate([-q[:, hq:], q[:, :hq]], axis=1)
    q_ref[...] = q * cq_ref[...] + qrot * sq_ref[...]
    hk = NKV * 32
    krot = jnp.concatenate([-k[:, hk:], k[:, :hk]], axis=1)
    k_ref[...] = k * ck_ref[...] + krot * sk_ref[...]
    v_ref[...] = v


def _attn_body(q_ref, k_ref, v_ref, o_ref):
    qb = pl.program_id(1)
    q = q_ref[0]
    k = k_ref[0]
    s = _dot_t(q, k) * (1.0 / math.sqrt(HD))
    row = qb * _BQ + lax.broadcasted_iota(jnp.int32, (_BQ, S), 0)
    col = lax.broadcasted_iota(jnp.int32, (_BQ, S), 1)
    s = jnp.where(col <= row, s, NEG)
    m = jnp.max(s, axis=1, keepdims=True)
    p = jnp.exp(s - m)
    p = p / jnp.sum(p, axis=1, keepdims=True)
    o_ref[0] = jnp.dot(p, v_ref[0], preferred_element_type=jnp.float32)


def _out_router_body(ctx_ref, h_ref, wo_ref, ln2_ref, gw_ref, gb_ref,
                     h1_ref, x2_ref, tw_ref, ti_ref):
    attn_out = _dot_t(ctx_ref[...], wo_ref[...])
    h1 = h_ref[...] + attn_out
    h1_ref[...] = h1
    var = jnp.mean(h1 * h1, axis=1, keepdims=True)
    x2 = (h1 * lax.rsqrt(var + EPS)) * ln2_ref[...]
    x2_ref[...] = x2
    logits = _dot_t(x2, gw_ref[...]) + gb_ref[...]
    mx = jnp.max(logits, axis=1, keepdims=True)
    ex = jnp.exp(logits - mx)
    probs = ex / jnp.sum(ex, axis=1, keepdims=True)
    idx = lax.broadcasted_iota(jnp.int32, probs.shape, 1)
    m1 = jnp.max(probs, axis=1, keepdims=True)
    c1 = jnp.where(probs == m1, idx, E)
    i1 = jnp.min(c1, axis=1, keepdims=True)
    p2 = jnp.where(idx == i1, -1.0, probs)
    m2 = jnp.max(p2, axis=1, keepdims=True)
    c2 = jnp.where(p2 == m2, idx, E)
    i2 = jnp.min(c2, axis=1, keepdims=True)
    tw_ref[...] = jnp.concatenate([m1, m2], axis=1)
    ti_ref[...] = jnp.concatenate([i1, i2], axis=1)


def _make_row_gather(nrows_out, chunk):
    b_per_w = nrows_out // _NW
    nchunk = b_per_w // chunk
    mesh = plsc.VectorSubcoreMesh(core_axis_name="c", subcore_axis_name="s")

    @functools.partial(
        pl.kernel, mesh=mesh,
        out_type=jax.ShapeDtypeStruct((nrows_out, H), jnp.float32),
        scratch_types=[
            pltpu.VMEM((b_per_w,), jnp.int32),
            pltpu.VMEM((chunk, H), jnp.float32),
            pltpu.SemaphoreType.DMA,
        ],
    )
    def gk(table_hbm, idx_hbm, out_hbm, idx_v, rows_v, sem):
        wid = lax.axis_index("s") * 2 + lax.axis_index("c")
        base = wid * b_per_w
        pltpu.sync_copy(idx_hbm.at[pl.ds(base, b_per_w)], idx_v)
        for c in range(nchunk):
            pltpu.async_copy(
                table_hbm.at[idx_v.at[pl.ds(c * chunk, chunk)]], rows_v, sem
            ).wait()
            pltpu.sync_copy(rows_v, out_hbm.at[pl.ds(base + c * chunk, chunk)])

    return gk


def _group_body(be_ref, nr_ref, xg_ref, w1_ref, w3_ref, w2_ref,
                o_ref, acc_ref, sem):
    i = pl.program_id(0)
    b = pl.program_id(1)
    ni = pl.num_programs(0)
    nb = pl.num_programs(1)

    @pl.when(b < nr_ref[0])
    def _compute():
        rows = pl.ds(b * _GBS, _GBS)
        x = xg_ref[rows, :]
        a1 = _dot_t(x, w1_ref[0])
        a3 = _dot_t(x, w3_ref[0])
        g = (a1 / (1.0 + jnp.exp(-a1))) * a3
        part = _dot_t(g, w2_ref[0])

        @pl.when(i == 0)
        def _init():
            acc_ref[rows, :] = part

        @pl.when(i > 0)
        def _acc():
            acc_ref[rows, :] += part

    @pl.when((i == ni - 1) & (b == nb - 1))
    def _flush():
        cp = pltpu.make_async_copy(acc_ref, o_ref, sem)
        cp.start()
        cp.wait()


def _combine_body(h1_ref, y0_ref, y1_ref, tw_ref, o_ref):
    w0 = tw_ref[:, 0:1]
    w1c = tw_ref[:, 1:2]
    o_ref[...] = h1_ref[...] + w0 * y0_ref[0] + w1c * y1_ref[0]


def kernel(h, ln1_w, ln2_w, wq, wk, wv, wo, gate_w, gate_b, w1, w2, w3):
    f32 = jnp.float32
    inv = 1.0 / (THETA ** (np.arange(0, HD, 2, dtype=np.float32) / HD))
    t = np.arange(S, dtype=np.float32)
    f_a = jnp.asarray(np.outer(t, inv), dtype=f32)
    cos_a, sin_a = jnp.cos(f_a), jnp.sin(f_a)
    cq = jnp.tile(cos_a, (1, NH * 2))
    sq = jnp.tile(sin_a, (1, NH * 2))
    ck = jnp.tile(cos_a, (1, NKV * 2))
    sk = jnp.tile(sin_a, (1, NKV * 2))

    def _perm(nh):
        base = np.arange(nh)[:, None] * HD + np.arange(32)[None, :]
        return np.concatenate([base.ravel(), (base + 32).ravel()])

    wq_p = wq[_perm(NH)]
    wk_p = wk[_perm(NKV)]

    nb1 = S // _BS1
    q, k, v = pl.pallas_call(
        _qkv_body,
        grid=(nb1,),
        in_specs=[
            pl.BlockSpec((_BS1, H), lambda i: (i, 0)),
            pl.BlockSpec((1, H), lambda i: (0, 0)),
            pl.BlockSpec((_BS1, NH * HD), lambda i: (i, 0)),
            pl.BlockSpec((_BS1, NH * HD), lambda i: (i, 0)),
            pl.BlockSpec((_BS1, NKV * HD), lambda i: (i, 0)),
            pl.BlockSpec((_BS1, NKV * HD), lambda i: (i, 0)),
            pl.BlockSpec((NH * HD, H), lambda i: (0, 0)),
            pl.BlockSpec((NKV * HD, H), lambda i: (0, 0)),
            pl.BlockSpec((NKV * HD, H), lambda i: (0, 0)),
        ],
        out_specs=[
            pl.BlockSpec((_BS1, NH * HD), lambda i: (i, 0)),
            pl.BlockSpec((_BS1, NKV * HD), lambda i: (i, 0)),
            pl.BlockSpec((_BS1, NKV * HD), lambda i: (i, 0)),
        ],
        out_shape=[
            jax.ShapeDtypeStruct((S, NH * HD), f32),
            jax.ShapeDtypeStruct((S, NKV * HD), f32),
            jax.ShapeDtypeStruct((S, NKV * HD), f32),
        ],
    )(h, ln1_w.reshape(1, H), cq, sq, ck, sk, wq_p, wk_p, wv)

    qh = q.reshape(S, 2, NH, 32).transpose(2, 0, 1, 3).reshape(NH, S, HD)
    kh = k.reshape(S, 2, NKV, 32).transpose(2, 0, 1, 3).reshape(NKV, S, HD)
    vh = v.reshape(S, NKV, HD).transpose(1, 0, 2)

    rep = NH // NKV
    ctx = pl.pallas_call(
        _attn_body,
        grid=(NH, S // _BQ),
        in_specs=[
            pl.BlockSpec((1, _BQ, HD), lambda hh, qb: (hh, qb, 0)),
            pl.BlockSpec((1, S, HD), lambda hh, qb: (hh // rep, 0, 0)),
            pl.BlockSpec((1, S, HD), lambda hh, qb: (hh // rep, 0, 0)),
        ],
        out_specs=pl.BlockSpec((1, _BQ, HD), lambda hh, qb: (hh, qb, 0)),
        out_shape=jax.ShapeDtypeStruct((NH, S, HD), f32),
    )(qh, kh, vh)

    ctx2 = ctx.transpose(1, 0, 2).reshape(S, NH * HD)

    nb3 = S // _BS3
    h1, x2, topw, topi = pl.pallas_call(
        _out_router_body,
        grid=(nb3,),
        in_specs=[
            pl.BlockSpec((_BS3, NH * HD), lambda i: (i, 0)),
            pl.BlockSpec((_BS3, H), lambda i: (i, 0)),
            pl.BlockSpec((H, NH * HD), lambda i: (0, 0)),
            pl.BlockSpec((1, H), lambda i: (0, 0)),
            pl.BlockSpec((E, H), lambda i: (0, 0)),
            pl.BlockSpec((1, E), lambda i: (0, 0)),
        ],
        out_specs=[
            pl.BlockSpec((_BS3, H), lambda i: (i, 0)),
            pl.BlockSpec((_BS3, H), lambda i: (i, 0)),
            pl.BlockSpec((_BS3, K), lambda i: (i, 0)),
            pl.BlockSpec((_BS3, K), lambda i: (i, 0)),
        ],
        out_shape=[
            jax.ShapeDtypeStruct((S, H), f32),
            jax.ShapeDtypeStruct((S, H), f32),
            jax.ShapeDtypeStruct((S, K), f32),
            jax.ShapeDtypeStruct((S, K), jnp.int32),
        ],
    )(ctx2, h, wo, ln2_w.reshape(1, H), gate_w, gate_b.reshape(1, E))

    i32 = jnp.int32
    sk = S * K
    e_flat = topi.reshape(sk)
    oh = (e_flat[:, None] == jnp.arange(E, dtype=i32)[None, :]).astype(i32)
    pref = jnp.cumsum(oh, axis=0)
    rank = jnp.sum(jnp.where(oh > 0, pref - 1, 0), axis=1)
    cnt = pref[-1]
    pcnt = ((cnt + _GBS - 1) // _GBS) * _GBS
    poff = jnp.concatenate([jnp.zeros((1,), i32),
                            jnp.cumsum(pcnt)[:-1].astype(i32)])
    slot = poff[e_flat] + rank
    row_id = jnp.zeros((_PMAX,), i32).at[slot].set(
        jnp.arange(sk, dtype=i32) // K)
    nblk = (pcnt // _GBS).astype(i32)
    block_expert = jnp.minimum(
        jnp.repeat(jnp.arange(E, dtype=i32), nblk, total_repeat_length=_NB),
        E - 1)
    nreal = jnp.sum(nblk).reshape(1)

    xg = _make_row_gather(_PMAX, 32)(x2, row_id)

    ni = I // _BI
    yg = pl.pallas_call(
        _group_body,
        grid_spec=pltpu.PrefetchScalarGridSpec(
            num_scalar_prefetch=2,
            grid=(ni, _NB),
            in_specs=[
                pl.BlockSpec((_PMAX, H), lambda i, b, be, nr: (0, 0)),
                pl.BlockSpec((1, _BI, H), lambda i, b, be, nr: (be[b], i, 0)),
                pl.BlockSpec((1, _BI, H), lambda i, b, be, nr: (be[b], i, 0)),
                pl.BlockSpec((1, H, _BI), lambda i, b, be, nr: (be[b], 0, i)),
            ],
            out_specs=pl.BlockSpec(memory_space=pl.ANY),
            scratch_shapes=[
                pltpu.VMEM((_PMAX, H), f32),
                pltpu.SemaphoreType.DMA,
            ],
        ),
        out_shape=jax.ShapeDtypeStruct((_PMAX, H), f32),
    )(block_expert, nreal, xg, w1, w3, w2)

    pcat = slot.reshape(S, K).transpose(1, 0).reshape(sk)
    ygc = _make_row_gather(sk, 32)(yg, pcat).reshape(K, S, H)

    out = pl.pallas_call(
        _combine_body,
        grid=(nb3,),
        in_specs=[
            pl.BlockSpec((_BS3, H), lambda i: (i, 0)),
            pl.BlockSpec((1, _BS3, H), lambda i: (0, i, 0)),
            pl.BlockSpec((1, _BS3, H), lambda i: (1, i, 0)),
            pl.BlockSpec((_BS3, K), lambda i: (i, 0)),
        ],
        out_specs=pl.BlockSpec((_BS3, H), lambda i: (i, 0)),
        out_shape=jax.ShapeDtypeStruct((S, H), f32),
    )(h1, ygc, ygc, topw)
    return out


# speedup vs baseline: 1.0841x; 1.0841x over previous
import functools
import math

import jax
import jax.numpy as jnp
import numpy as np
from jax import lax
from jax.experimental import pallas as pl
from jax.experimental.pallas import tpu as pltpu
from jax.experimental.pallas import tpu_sc as plsc

S = 2048
H = 1024
NH = 16
NKV = 8
HD = 64
I = 3584
E = 8
K = 2
THETA = 10000.0
EPS = 1e-06
NEG = float(jnp.finfo(jnp.float32).min)

_BS1 = 256
_BQ = 256
_BS3 = 512
_BI = 512
_GBS = 128
_PMAX = S * K + E * _GBS
_NB = _PMAX // _GBS
_NW = 32


def _dot_t(a, b):
    return lax.dot_general(a, b, (((1,), (1,)), ((), ())),
                           preferred_element_type=jnp.float32)


def _qkv_body(h_ref, ln1_ref, wq_ref, wk_ref, wv_ref, q_ref, k_ref, v_ref):
    x = h_ref[...]
    var = jnp.mean(x * x, axis=1, keepdims=True)
    xn = (x * lax.rsqrt(var + EPS)) * ln1_ref[...]
    q_ref[...] = _dot_t(xn, wq_ref[...])
    k_ref[...] = _dot_t(xn, wk_ref[...])
    v_ref[...] = _dot_t(xn, wv_ref[...])


def _rope(x, c, s):
    half = x.shape[1] // 2
    xr = jnp.concatenate([-x[:, half:], x[:, :half]], axis=1)
    return x * c + xr * s


_NHP = 4


def _attn_body(q_ref, k_ref, v_ref, cq_ref, sq_ref, ck_ref, sk_ref,
               o_ref, kr_ref):
    qb = pl.program_id(1)
    c64, s64 = ck_ref[...], sk_ref[...]
    kr_ref[...] = jnp.concatenate(
        [_rope(k_ref[:, :HD], c64, s64), _rope(k_ref[:, HD:], c64, s64)],
        axis=1)
    scale = 1.0 / math.sqrt(HD)
    row = lax.broadcasted_iota(jnp.int32, (_BQ, _BQ), 0)
    col = lax.broadcasted_iota(jnp.int32, (_BQ, _BQ), 1)

    outs = []
    for hh in range(_NHP):
        q = _rope(q_ref[:, hh * HD:(hh + 1) * HD], cq_ref[...], sq_ref[...])
        kvc = (hh // 2) * HD

        def chunk(c, carry, q=q, kvc=kvc):
            m, l, acc = carry
            kc = kr_ref[pl.ds(c * _BQ, _BQ), kvc:kvc + HD]
            s = _dot_t(q, kc) * scale
            s = jnp.where((qb - c) * _BQ + row >= col, s, NEG)
            m2 = jnp.maximum(m, jnp.max(s, axis=1, keepdims=True))
            p = jnp.exp(s - m2)
            corr = jnp.exp(m - m2)
            l2 = l * corr + jnp.sum(p, axis=1, keepdims=True)
            vc = v_ref[pl.ds(c * _BQ, _BQ), kvc:kvc + HD]
            acc2 = acc * corr + jnp.dot(p, vc,
                                        preferred_element_type=jnp.float32)
            return m2, l2, acc2

        m0 = jnp.full((_BQ, 1), NEG, dtype=jnp.float32)
        l0 = jnp.zeros((_BQ, 1), dtype=jnp.float32)
        a0 = jnp.zeros((_BQ, HD), dtype=jnp.float32)
        m, l, acc = lax.fori_loop(0, qb + 1, chunk, (m0, l0, a0))
        outs.append(acc / l)
    o_ref[...] = jnp.concatenate(outs, axis=1)


def _out_router_body(ctx_ref, h_ref, wo_ref, ln2_ref, gw_ref, gb_ref,
                     h1_ref, x2_ref, tw_ref, ti_ref):
    attn_out = _dot_t(ctx_ref[...], wo_ref[...])
    h1 = h_ref[...] + attn_out
    h1_ref[...] = h1
    var = jnp.mean(h1 * h1, axis=1, keepdims=True)
    x2 = (h1 * lax.rsqrt(var + EPS)) * ln2_ref[...]
    x2_ref[...] = x2
    logits = _dot_t(x2, gw_ref[...]) + gb_ref[...]
    mx = jnp.max(logits, axis=1, keepdims=True)
    ex = jnp.exp(logits - mx)
    probs = ex / jnp.sum(ex, axis=1, keepdims=True)
    idx = lax.broadcasted_iota(jnp.int32, probs.shape, 1)
    m1 = jnp.max(probs, axis=1, keepdims=True)
    c1 = jnp.where(probs == m1, idx, E)
    i1 = jnp.min(c1, axis=1, keepdims=True)
    p2 = jnp.where(idx == i1, -1.0, probs)
    m2 = jnp.max(p2, axis=1, keepdims=True)
    c2 = jnp.where(p2 == m2, idx, E)
    i2 = jnp.min(c2, axis=1, keepdims=True)
    tw_ref[...] = jnp.concatenate([m1, m2], axis=1)
    ti_ref[...] = jnp.concatenate([i1, i2], axis=1)


def _make_row_gather(nrows_out, chunk):
    b_per_w = nrows_out // _NW
    nchunk = b_per_w // chunk
    mesh = plsc.VectorSubcoreMesh(core_axis_name="c", subcore_axis_name="s")

    @functools.partial(
        pl.kernel, mesh=mesh,
        out_type=jax.ShapeDtypeStruct((nrows_out, H), jnp.float32),
        scratch_types=[
            pltpu.VMEM((b_per_w,), jnp.int32),
            pltpu.VMEM((chunk, H), jnp.float32),
            pltpu.SemaphoreType.DMA,
        ],
    )
    def gk(table_hbm, idx_hbm, out_hbm, idx_v, rows_v, sem):
        wid = lax.axis_index("s") * 2 + lax.axis_index("c")
        base = wid * b_per_w
        pltpu.sync_copy(idx_hbm.at[pl.ds(base, b_per_w)], idx_v)
        for c in range(nchunk):
            pltpu.async_copy(
                table_hbm.at[idx_v.at[pl.ds(c * chunk, chunk)]], rows_v, sem
            ).wait()
            pltpu.sync_copy(rows_v, out_hbm.at[pl.ds(base + c * chunk, chunk)])

    return gk


def _group_body(be_ref, nr_ref, xg_ref, w1_ref, w3_ref, w2_ref,
                o_ref, acc_ref, sem):
    i = pl.program_id(0)
    b = pl.program_id(1)
    ni = pl.num_programs(0)
    nb = pl.num_programs(1)

    @pl.when(b < nr_ref[0])
    def _compute():
        rows = pl.ds(b * _GBS, _GBS)
        x = xg_ref[rows, :]
        a1 = _dot_t(x, w1_ref[0])
        a3 = _dot_t(x, w3_ref[0])
        g = (a1 / (1.0 + jnp.exp(-a1))) * a3
        part = _dot_t(g, w2_ref[0])

        @pl.when(i == 0)
        def _init():
            acc_ref[rows, :] = part

        @pl.when(i > 0)
        def _acc():
            acc_ref[rows, :] += part

    @pl.when((i == ni - 1) & (b == nb - 1))
    def _flush():
        cp = pltpu.make_async_copy(acc_ref, o_ref, sem)
        cp.start()
        cp.wait()


def _combine_body(h1_ref, y0_ref, y1_ref, tw_ref, o_ref):
    w0 = tw_ref[:, 0:1]
    w1c = tw_ref[:, 1:2]
    o_ref[...] = h1_ref[...] + w0 * y0_ref[0] + w1c * y1_ref[0]


def kernel(h, ln1_w, ln2_w, wq, wk, wv, wo, gate_w, gate_b, w1, w2, w3):
    f32 = jnp.float32
    inv = 1.0 / (THETA ** (np.arange(0, HD, 2, dtype=np.float32) / HD))
    t = np.arange(S, dtype=np.float32)
    f_a = np.concatenate([np.outer(t, inv)] * 2, axis=1)
    cos64 = jnp.asarray(np.cos(f_a), dtype=f32)
    sin64 = jnp.asarray(np.sin(f_a), dtype=f32)

    nb1 = S // _BS1
    q, k, v = pl.pallas_call(
        _qkv_body,
        grid=(nb1,),
        in_specs=[
            pl.BlockSpec((_BS1, H), lambda i: (i, 0)),
            pl.BlockSpec((1, H), lambda i: (0, 0)),
            pl.BlockSpec((NH * HD, H), lambda i: (0, 0)),
            pl.BlockSpec((NKV * HD, H), lambda i: (0, 0)),
            pl.BlockSpec((NKV * HD, H), lambda i: (0, 0)),
        ],
        out_specs=[
            pl.BlockSpec((_BS1, NH * HD), lambda i: (i, 0)),
            pl.BlockSpec((_BS1, NKV * HD), lambda i: (i, 0)),
            pl.BlockSpec((_BS1, NKV * HD), lambda i: (i, 0)),
        ],
        out_shape=[
            jax.ShapeDtypeStruct((S, NH * HD), f32),
            jax.ShapeDtypeStruct((S, NKV * HD), f32),
            jax.ShapeDtypeStruct((S, NKV * HD), f32),
        ],
    )(h, ln1_w.reshape(1, H), wq, wk, wv)

    ctx2 = pl.pallas_call(
        _attn_body,
        grid=(NH // _NHP, S // _BQ),
        in_specs=[
            pl.BlockSpec((_BQ, _NHP * HD), lambda g, qb: (qb, g)),
            pl.BlockSpec((S, 2 * HD), lambda g, qb: (0, g)),
            pl.BlockSpec((S, 2 * HD), lambda g, qb: (0, g)),
            pl.BlockSpec((_BQ, HD), lambda g, qb: (qb, 0)),
            pl.BlockSpec((_BQ, HD), lambda g, qb: (qb, 0)),
            pl.BlockSpec((S, HD), lambda g, qb: (0, 0)),
            pl.BlockSpec((S, HD), lambda g, qb: (0, 0)),
        ],
        out_specs=pl.BlockSpec((_BQ, _NHP * HD), lambda g, qb: (qb, g)),
        out_shape=jax.ShapeDtypeStruct((S, NH * HD), f32),
        scratch_shapes=[pltpu.VMEM((S, 2 * HD), f32)],
    )(q, k, v, cos64, sin64, cos64, sin64)

    nb3 = S // _BS3
    h1, x2, topw, topi = pl.pallas_call(
        _out_router_body,
        grid=(nb3,),
        in_specs=[
            pl.BlockSpec((_BS3, NH * HD), lambda i: (i, 0)),
            pl.BlockSpec((_BS3, H), lambda i: (i, 0)),
            pl.BlockSpec((H, NH * HD), lambda i: (0, 0)),
            pl.BlockSpec((1, H), lambda i: (0, 0)),
            pl.BlockSpec((E, H), lambda i: (0, 0)),
            pl.BlockSpec((1, E), lambda i: (0, 0)),
        ],
        out_specs=[
            pl.BlockSpec((_BS3, H), lambda i: (i, 0)),
            pl.BlockSpec((_BS3, H), lambda i: (i, 0)),
            pl.BlockSpec((_BS3, K), lambda i: (i, 0)),
            pl.BlockSpec((_BS3, K), lambda i: (i, 0)),
        ],
        out_shape=[
            jax.ShapeDtypeStruct((S, H), f32),
            jax.ShapeDtypeStruct((S, H), f32),
            jax.ShapeDtypeStruct((S, K), f32),
            jax.ShapeDtypeStruct((S, K), jnp.int32),
        ],
    )(ctx2, h, wo, ln2_w.reshape(1, H), gate_w, gate_b.reshape(1, E))

    i32 = jnp.int32
    sk = S * K
    e_flat = topi.reshape(sk)
    oh = (e_flat[:, None] == jnp.arange(E, dtype=i32)[None, :]).astype(i32)
    pref = jnp.cumsum(oh, axis=0)
    rank = jnp.sum(jnp.where(oh > 0, pref - 1, 0), axis=1)
    cnt = pref[-1]
    pcnt = ((cnt + _GBS - 1) // _GBS) * _GBS
    poff = jnp.concatenate([jnp.zeros((1,), i32),
                            jnp.cumsum(pcnt)[:-1].astype(i32)])
    slot = poff[e_flat] + rank
    row_id = jnp.zeros((_PMAX,), i32).at[slot].set(
        jnp.arange(sk, dtype=i32) // K)
    nblk = (pcnt // _GBS).astype(i32)
    block_expert = jnp.minimum(
        jnp.repeat(jnp.arange(E, dtype=i32), nblk, total_repeat_length=_NB),
        E - 1)
    nreal = jnp.sum(nblk).reshape(1)

    xg = _make_row_gather(_PMAX, 32)(x2, row_id)

    ni = I // _BI
    yg = pl.pallas_call(
        _group_body,
        grid_spec=pltpu.PrefetchScalarGridSpec(
            num_scalar_prefetch=2,
            grid=(ni, _NB),
            in_specs=[
                pl.BlockSpec((_PMAX, H), lambda i, b, be, nr: (0, 0)),
                pl.BlockSpec((1, _BI, H), lambda i, b, be, nr: (be[b], i, 0)),
                pl.BlockSpec((1, _BI, H), lambda i, b, be, nr: (be[b], i, 0)),
                pl.BlockSpec((1, H, _BI), lambda i, b, be, nr: (be[b], 0, i)),
            ],
            out_specs=pl.BlockSpec(memory_space=pl.ANY),
            scratch_shapes=[
                pltpu.VMEM((_PMAX, H), f32),
                pltpu.SemaphoreType.DMA,
            ],
        ),
        out_shape=jax.ShapeDtypeStruct((_PMAX, H), f32),
    )(block_expert, nreal, xg, w1, w3, w2)

    pcat = slot.reshape(S, K).transpose(1, 0).reshape(sk)
    ygc = _make_row_gather(sk, 32)(yg, pcat).reshape(K, S, H)

    out = pl.pallas_call(
        _combine_body,
        grid=(nb3,),
        in_specs=[
            pl.BlockSpec((_BS3, H), lambda i: (i, 0)),
            pl.BlockSpec((1, _BS3, H), lambda i: (0, i, 0)),
            pl.BlockSpec((1, _BS3, H), lambda i: (1, i, 0)),
            pl.BlockSpec((_BS3, K), lambda i: (i, 0)),
        ],
        out_specs=pl.BlockSpec((_BS3, H), lambda i: (i, 0)),
        out_shape=jax.ShapeDtypeStruct((S, H), f32),
    )(h1, ygc, ygc, topw)
    return out


# speedup vs baseline: 1.0864x; 1.0021x over previous
import functools
import math

import jax
import jax.numpy as jnp
import numpy as np
from jax import lax
from jax.experimental import pallas as pl
from jax.experimental.pallas import tpu as pltpu
from jax.experimental.pallas import tpu_sc as plsc

S = 2048
H = 1024
NH = 16
NKV = 8
HD = 64
I = 3584
E = 8
K = 2
THETA = 10000.0
EPS = 1e-06
NEG = float(jnp.finfo(jnp.float32).min)

_BS1 = 256
_BQ = 256
_BS3 = 512
_BI = 512
_GBS = 128
_PMAX = S * K + E * _GBS
_NB = _PMAX // _GBS
_NW = 32


def _dot_t(a, b):
    return lax.dot_general(a, b, (((1,), (1,)), ((), ())),
                           preferred_element_type=jnp.float32)


def _qkv_body(h_ref, ln1_ref, wq_ref, wk_ref, wv_ref, q_ref, k_ref, v_ref):
    x = h_ref[...]
    var = jnp.mean(x * x, axis=1, keepdims=True)
    xn = (x * lax.rsqrt(var + EPS)) * ln1_ref[...]
    q_ref[...] = _dot_t(xn, wq_ref[...])
    k_ref[...] = _dot_t(xn, wk_ref[...])
    v_ref[...] = _dot_t(xn, wv_ref[...])


def _rope(x, c, s):
    half = x.shape[1] // 2
    xr = jnp.concatenate([-x[:, half:], x[:, :half]], axis=1)
    return x * c + xr * s


_NHP = 4


def _attn_body(q_ref, k_ref, v_ref, cq_ref, sq_ref, ck_ref, sk_ref,
               o_ref, kr_ref):
    qb = pl.program_id(1)
    c64, s64 = ck_ref[...], sk_ref[...]
    kr_ref[...] = jnp.concatenate(
        [_rope(k_ref[:, :HD], c64, s64), _rope(k_ref[:, HD:], c64, s64)],
        axis=1)
    scale = 1.0 / math.sqrt(HD)
    row = lax.broadcasted_iota(jnp.int32, (_BQ, _BQ), 0)
    col = lax.broadcasted_iota(jnp.int32, (_BQ, _BQ), 1)

    outs = []
    for hh in range(_NHP):
        q = _rope(q_ref[:, hh * HD:(hh + 1) * HD], cq_ref[...], sq_ref[...])
        kvc = (hh // 2) * HD

        def chunk(c, carry, q=q, kvc=kvc):
            m, l, acc = carry
            kc = kr_ref[pl.ds(c * _BQ, _BQ), kvc:kvc + HD]
            s = _dot_t(q, kc) * scale
            s = jnp.where((qb - c) * _BQ + row >= col, s, NEG)
            m2 = jnp.maximum(m, jnp.max(s, axis=1, keepdims=True))
            p = jnp.exp(s - m2)
            corr = jnp.exp(m - m2)
            l2 = l * corr + jnp.sum(p, axis=1, keepdims=True)
            vc = v_ref[pl.ds(c * _BQ, _BQ), kvc:kvc + HD]
            acc2 = acc * corr + jnp.dot(p, vc,
                                        preferred_element_type=jnp.float32)
            return m2, l2, acc2

        m0 = jnp.full((_BQ, 1), NEG, dtype=jnp.float32)
        l0 = jnp.zeros((_BQ, 1), dtype=jnp.float32)
        a0 = jnp.zeros((_BQ, HD), dtype=jnp.float32)
        m, l, acc = lax.fori_loop(0, qb + 1, chunk, (m0, l0, a0))
        outs.append(acc / l)
    o_ref[...] = jnp.concatenate(outs, axis=1)


def _out_router_body(ctx_ref, h_ref, wo_ref, ln2_ref, gw_ref, gb_ref,
                     h1_ref, x2_ref, tw_ref, ti_ref):
    attn_out = _dot_t(ctx_ref[...], wo_ref[...])
    h1 = h_ref[...] + attn_out
    h1_ref[...] = h1
    var = jnp.mean(h1 * h1, axis=1, keepdims=True)
    x2 = (h1 * lax.rsqrt(var + EPS)) * ln2_ref[...]
    x2_ref[...] = x2
    logits = _dot_t(x2, gw_ref[...]) + gb_ref[...]
    mx = jnp.max(logits, axis=1, keepdims=True)
    ex = jnp.exp(logits - mx)
    probs = ex / jnp.sum(ex, axis=1, keepdims=True)
    idx = lax.broadcasted_iota(jnp.int32, probs.shape, 1)
    m1 = jnp.max(probs, axis=1, keepdims=True)
    c1 = jnp.where(probs == m1, idx, E)
    i1 = jnp.min(c1, axis=1, keepdims=True)
    p2 = jnp.where(idx == i1, -1.0, probs)
    m2 = jnp.max(p2, axis=1, keepdims=True)
    c2 = jnp.where(p2 == m2, idx, E)
    i2 = jnp.min(c2, axis=1, keepdims=True)
    tw_ref[...] = jnp.concatenate([m1, m2], axis=1)
    ti_ref[...] = jnp.concatenate([i1, i2], axis=1)


def _make_row_gather(nrows_out, chunk):
    b_per_w = nrows_out // _NW
    nchunk = b_per_w // chunk
    mesh = plsc.VectorSubcoreMesh(core_axis_name="c", subcore_axis_name="s")

    @functools.partial(
        pl.kernel, mesh=mesh,
        out_type=jax.ShapeDtypeStruct((nrows_out, H), jnp.float32),
        scratch_types=[
            pltpu.VMEM((b_per_w,), jnp.int32),
            pltpu.VMEM((chunk, H), jnp.float32),
            pltpu.VMEM((chunk, H), jnp.float32),
            pltpu.SemaphoreType.DMA,
            pltpu.SemaphoreType.DMA,
        ],
    )
    def gk(table_hbm, idx_hbm, out_hbm, idx_v, rows_a, rows_b, gsem, wsem):
        wid = lax.axis_index("s") * 2 + lax.axis_index("c")
        base = wid * b_per_w
        bufs = (rows_a, rows_b)
        pltpu.sync_copy(idx_hbm.at[pl.ds(base, b_per_w)], idx_v)

        def gather(c):
            return pltpu.async_copy(
                table_hbm.at[idx_v.at[pl.ds(c * chunk, chunk)]],
                bufs[c % 2], gsem)

        def write(c):
            return pltpu.async_copy(
                bufs[c % 2], out_hbm.at[pl.ds(base + c * chunk, chunk)], wsem)

        g = gather(0)
        writes = []
        for c in range(nchunk):
            g.wait()
            writes.append(write(c))
            if c + 1 < nchunk:
                if c >= 1:
                    writes[c - 1].wait()
                g = gather(c + 1)
        if nchunk >= 2:
            writes[nchunk - 2].wait()
        writes[nchunk - 1].wait()

    return gk


def _group_body(be_ref, nr_ref, xg_ref, w1_ref, w3_ref, w2_ref,
                o_ref, acc_ref, sem):
    i = pl.program_id(0)
    b = pl.program_id(1)
    ni = pl.num_programs(0)
    nb = pl.num_programs(1)

    @pl.when(b < nr_ref[0])
    def _compute():
        rows = pl.ds(b * _GBS, _GBS)
        x = xg_ref[rows, :]
        a1 = _dot_t(x, w1_ref[0])
        a3 = _dot_t(x, w3_ref[0])
        g = (a1 / (1.0 + jnp.exp(-a1))) * a3
        part = _dot_t(g, w2_ref[0])

        @pl.when(i == 0)
        def _init():
            acc_ref[rows, :] = part

        @pl.when(i > 0)
        def _acc():
            acc_ref[rows, :] += part

    @pl.when((i == ni - 1) & (b == nb - 1))
    def _flush():
        cp = pltpu.make_async_copy(acc_ref, o_ref, sem)
        cp.start()
        cp.wait()


def _combine_body(h1_ref, y0_ref, y1_ref, tw_ref, o_ref):
    w0 = tw_ref[:, 0:1]
    w1c = tw_ref[:, 1:2]
    o_ref[...] = h1_ref[...] + w0 * y0_ref[0] + w1c * y1_ref[0]


def kernel(h, ln1_w, ln2_w, wq, wk, wv, wo, gate_w, gate_b, w1, w2, w3):
    f32 = jnp.float32
    inv = 1.0 / (THETA ** (np.arange(0, HD, 2, dtype=np.float32) / HD))
    t = np.arange(S, dtype=np.float32)
    f_a = np.concatenate([np.outer(t, inv)] * 2, axis=1)
    cos64 = jnp.asarray(np.cos(f_a), dtype=f32)
    sin64 = jnp.asarray(np.sin(f_a), dtype=f32)

    nb1 = S // _BS1
    q, k, v = pl.pallas_call(
        _qkv_body,
        grid=(nb1,),
        in_specs=[
            pl.BlockSpec((_BS1, H), lambda i: (i, 0)),
            pl.BlockSpec((1, H), lambda i: (0, 0)),
            pl.BlockSpec((NH * HD, H), lambda i: (0, 0)),
            pl.BlockSpec((NKV * HD, H), lambda i: (0, 0)),
            pl.BlockSpec((NKV * HD, H), lambda i: (0, 0)),
        ],
        out_specs=[
            pl.BlockSpec((_BS1, NH * HD), lambda i: (i, 0)),
            pl.BlockSpec((_BS1, NKV * HD), lambda i: (i, 0)),
            pl.BlockSpec((_BS1, NKV * HD), lambda i: (i, 0)),
        ],
        out_shape=[
            jax.ShapeDtypeStruct((S, NH * HD), f32),
            jax.ShapeDtypeStruct((S, NKV * HD), f32),
            jax.ShapeDtypeStruct((S, NKV * HD), f32),
        ],
    )(h, ln1_w.reshape(1, H), wq, wk, wv)

    ctx2 = pl.pallas_call(
        _attn_body,
        grid=(NH // _NHP, S // _BQ),
        in_specs=[
            pl.BlockSpec((_BQ, _NHP * HD), lambda g, qb: (qb, g)),
            pl.BlockSpec((S, 2 * HD), lambda g, qb: (0, g)),
            pl.BlockSpec((S, 2 * HD), lambda g, qb: (0, g)),
            pl.BlockSpec((_BQ, HD), lambda g, qb: (qb, 0)),
            pl.BlockSpec((_BQ, HD), lambda g, qb: (qb, 0)),
            pl.BlockSpec((S, HD), lambda g, qb: (0, 0)),
            pl.BlockSpec((S, HD), lambda g, qb: (0, 0)),
        ],
        out_specs=pl.BlockSpec((_BQ, _NHP * HD), lambda g, qb: (qb, g)),
        out_shape=jax.ShapeDtypeStruct((S, NH * HD), f32),
        scratch_shapes=[pltpu.VMEM((S, 2 * HD), f32)],
    )(q, k, v, cos64, sin64, cos64, sin64)

    nb3 = S // _BS3
    h1, x2, topw, topi = pl.pallas_call(
        _out_router_body,
        grid=(nb3,),
        in_specs=[
            pl.BlockSpec((_BS3, NH * HD), lambda i: (i, 0)),
            pl.BlockSpec((_BS3, H), lambda i: (i, 0)),
            pl.BlockSpec((H, NH * HD), lambda i: (0, 0)),
            pl.BlockSpec((1, H), lambda i: (0, 0)),
            pl.BlockSpec((E, H), lambda i: (0, 0)),
            pl.BlockSpec((1, E), lambda i: (0, 0)),
        ],
        out_specs=[
            pl.BlockSpec((_BS3, H), lambda i: (i, 0)),
            pl.BlockSpec((_BS3, H), lambda i: (i, 0)),
            pl.BlockSpec((_BS3, K), lambda i: (i, 0)),
            pl.BlockSpec((_BS3, K), lambda i: (i, 0)),
        ],
        out_shape=[
            jax.ShapeDtypeStruct((S, H), f32),
            jax.ShapeDtypeStruct((S, H), f32),
            jax.ShapeDtypeStruct((S, K), f32),
            jax.ShapeDtypeStruct((S, K), jnp.int32),
        ],
    )(ctx2, h, wo, ln2_w.reshape(1, H), gate_w, gate_b.reshape(1, E))

    i32 = jnp.int32
    sk = S * K
    e_flat = topi.reshape(sk)
    oh = (e_flat[:, None] == jnp.arange(E, dtype=i32)[None, :]).astype(i32)
    pref = jnp.cumsum(oh, axis=0)
    rank = jnp.sum(jnp.where(oh > 0, pref - 1, 0), axis=1)
    cnt = pref[-1]
    pcnt = ((cnt + _GBS - 1) // _GBS) * _GBS
    poff = jnp.concatenate([jnp.zeros((1,), i32),
                            jnp.cumsum(pcnt)[:-1].astype(i32)])
    slot = poff[e_flat] + rank
    row_id = jnp.zeros((_PMAX,), i32).at[slot].set(
        jnp.arange(sk, dtype=i32) // K)
    nblk = (pcnt // _GBS).astype(i32)
    block_expert = jnp.minimum(
        jnp.repeat(jnp.arange(E, dtype=i32), nblk, total_repeat_length=_NB),
        E - 1)
    nreal = jnp.sum(nblk).reshape(1)

    xg = _make_row_gather(_PMAX, 32)(x2, row_id)

    ni = I // _BI
    yg = pl.pallas_call(
        _group_body,
        grid_spec=pltpu.PrefetchScalarGridSpec(
            num_scalar_prefetch=2,
            grid=(ni, _NB),
            in_specs=[
                pl.BlockSpec((_PMAX, H), lambda i, b, be, nr: (0, 0)),
                pl.BlockSpec((1, _BI, H), lambda i, b, be, nr: (be[b], i, 0)),
                pl.BlockSpec((1, _BI, H), lambda i, b, be, nr: (be[b], i, 0)),
                pl.BlockSpec((1, H, _BI), lambda i, b, be, nr: (be[b], 0, i)),
            ],
            out_specs=pl.BlockSpec(memory_space=pl.ANY),
            scratch_shapes=[
                pltpu.VMEM((_PMAX, H), f32),
                pltpu.SemaphoreType.DMA,
            ],
        ),
        out_shape=jax.ShapeDtypeStruct((_PMAX, H), f32),
    )(block_expert, nreal, xg, w1, w3, w2)

    pcat = slot.reshape(S, K).transpose(1, 0).reshape(sk)
    ygc = _make_row_gather(sk, 32)(yg, pcat).reshape(K, S, H)

    out = pl.pallas_call(
        _combine_body,
        grid=(nb3,),
        in_specs=[
            pl.BlockSpec((_BS3, H), lambda i: (i, 0)),
            pl.BlockSpec((1, _BS3, H), lambda i: (0, i, 0)),
            pl.BlockSpec((1, _BS3, H), lambda i: (1, i, 0)),
            pl.BlockSpec((_BS3, K), lambda i: (i, 0)),
        ],
        out_specs=pl.BlockSpec((_BS3, H), lambda i: (i, 0)),
        out_shape=jax.ShapeDtypeStruct((S, H), f32),
    )(h1, ygc, ygc, topw)
    return out


# speedup vs baseline: 1.2009x; 1.1055x over previous
import functools
import math

import jax
import jax.numpy as jnp
import numpy as np
from jax import lax
from jax.experimental import pallas as pl
from jax.experimental.pallas import tpu as pltpu
from jax.experimental.pallas import tpu_sc as plsc

S = 2048
H = 1024
NH = 16
NKV = 8
HD = 64
I = 3584
E = 8
K = 2
THETA = 10000.0
EPS = 1e-06
NEG = float(jnp.finfo(jnp.float32).min)

_BS1 = 256
_BQ = 256
_BS3 = 512
_BI = 512
_GBS = 128
_BID = 256
_PMAX = S * K + E * _GBS
_NB = _PMAX // _GBS
_NW = 32


def _dot_t(a, b):
    return lax.dot_general(a, b, (((1,), (1,)), ((), ())),
                           preferred_element_type=jnp.float32)


def _qkv_body(h_ref, ln1_ref, wq_ref, wk_ref, wv_ref, q_ref, k_ref, v_ref):
    x = h_ref[...]
    var = jnp.mean(x * x, axis=1, keepdims=True)
    xn = (x * lax.rsqrt(var + EPS)) * ln1_ref[...]
    q_ref[...] = _dot_t(xn, wq_ref[...])
    k_ref[...] = _dot_t(xn, wk_ref[...])
    v_ref[...] = _dot_t(xn, wv_ref[...])


def _rope(x, c, s):
    half = x.shape[1] // 2
    xr = jnp.concatenate([-x[:, half:], x[:, :half]], axis=1)
    return x * c + xr * s


_NHP = 4


def _attn_body(q_ref, k_ref, v_ref, cq_ref, sq_ref, ck_ref, sk_ref,
               o_ref, kr_ref):
    qb = pl.program_id(1)
    c64, s64 = ck_ref[...], sk_ref[...]
    kr_ref[...] = jnp.concatenate(
        [_rope(k_ref[:, :HD], c64, s64), _rope(k_ref[:, HD:], c64, s64)],
        axis=1)
    scale = 1.0 / math.sqrt(HD)
    row = lax.broadcasted_iota(jnp.int32, (_BQ, _BQ), 0)
    col = lax.broadcasted_iota(jnp.int32, (_BQ, _BQ), 1)

    outs = []
    for hh in range(_NHP):
        q = _rope(q_ref[:, hh * HD:(hh + 1) * HD], cq_ref[...], sq_ref[...])
        kvc = (hh // 2) * HD

        def chunk(c, carry, q=q, kvc=kvc):
            m, l, acc = carry
            kc = kr_ref[pl.ds(c * _BQ, _BQ), kvc:kvc + HD]
            s = _dot_t(q, kc) * scale
            s = jnp.where((qb - c) * _BQ + row >= col, s, NEG)
            m2 = jnp.maximum(m, jnp.max(s, axis=1, keepdims=True))
            p = jnp.exp(s - m2)
            corr = jnp.exp(m - m2)
            l2 = l * corr + jnp.sum(p, axis=1, keepdims=True)
            vc = v_ref[pl.ds(c * _BQ, _BQ), kvc:kvc + HD]
            acc2 = acc * corr + jnp.dot(p, vc,
                                        preferred_element_type=jnp.float32)
            return m2, l2, acc2

        m0 = jnp.full((_BQ, 1), NEG, dtype=jnp.float32)
        l0 = jnp.zeros((_BQ, 1), dtype=jnp.float32)
        a0 = jnp.zeros((_BQ, HD), dtype=jnp.float32)
        m, l, acc = lax.fori_loop(0, qb + 1, chunk, (m0, l0, a0))
        outs.append(acc / l)
    o_ref[...] = jnp.concatenate(outs, axis=1)


def _out_router_body(ctx_ref, h_ref, wo_ref, ln2_ref, gw_ref, gb_ref,
                     h1_ref, x2_ref, pw_ref):
    attn_out = _dot_t(ctx_ref[...], wo_ref[...])
    h1 = h_ref[...] + attn_out
    h1_ref[...] = h1
    var = jnp.mean(h1 * h1, axis=1, keepdims=True)
    x2 = (h1 * lax.rsqrt(var + EPS)) * ln2_ref[...]
    x2_ref[...] = x2
    logits = _dot_t(x2, gw_ref[...]) + gb_ref[...]
    mx = jnp.max(logits, axis=1, keepdims=True)
    ex = jnp.exp(logits - mx)
    probs = ex / jnp.sum(ex, axis=1, keepdims=True)
    idx = lax.broadcasted_iota(jnp.int32, probs.shape, 1)
    m1 = jnp.max(probs, axis=1, keepdims=True)
    c1 = jnp.where(probs == m1, idx, E)
    i1 = jnp.min(c1, axis=1, keepdims=True)
    p2 = jnp.where(idx == i1, -1.0, probs)
    m2 = jnp.max(p2, axis=1, keepdims=True)
    c2 = jnp.where(p2 == m2, idx, E)
    i2 = jnp.min(c2, axis=1, keepdims=True)
    pw_ref[...] = jnp.where(idx == i1, m1, jnp.where(idx == i2, m2, 0.0))


def _moe_body(x2_ref, h1_ref, pw_ref, w1_ref, w3_ref, w2_ref, o_ref):
    e = pl.program_id(0)
    i = pl.program_id(1)
    x2 = x2_ref[...]
    a1 = _dot_t(x2, w1_ref[0])
    a3 = _dot_t(x2, w3_ref[0])
    g = (a1 / (1.0 + jnp.exp(-a1))) * a3
    part = _dot_t(g, w2_ref[0])
    sel = lax.broadcasted_iota(jnp.int32, (S, E), 1) == e
    col = jnp.sum(jnp.where(sel, pw_ref[...], 0.0), axis=1, keepdims=True)
    contrib = col * part

    @pl.when((e == 0) & (i == 0))
    def _init():
        o_ref[...] = h1_ref[...] + contrib

    @pl.when((e > 0) | (i > 0))
    def _acc():
        o_ref[...] += contrib


def kernel(h, ln1_w, ln2_w, wq, wk, wv, wo, gate_w, gate_b, w1, w2, w3):
    f32 = jnp.float32
    inv = 1.0 / (THETA ** (np.arange(0, HD, 2, dtype=np.float32) / HD))
    t = np.arange(S, dtype=np.float32)
    f_a = np.concatenate([np.outer(t, inv)] * 2, axis=1)
    cos64 = jnp.asarray(np.cos(f_a), dtype=f32)
    sin64 = jnp.asarray(np.sin(f_a), dtype=f32)

    nb1 = S // _BS1
    q, k, v = pl.pallas_call(
        _qkv_body,
        grid=(nb1,),
        in_specs=[
            pl.BlockSpec((_BS1, H), lambda i: (i, 0)),
            pl.BlockSpec((1, H), lambda i: (0, 0)),
            pl.BlockSpec((NH * HD, H), lambda i: (0, 0)),
            pl.BlockSpec((NKV * HD, H), lambda i: (0, 0)),
            pl.BlockSpec((NKV * HD, H), lambda i: (0, 0)),
        ],
        out_specs=[
            pl.BlockSpec((_BS1, NH * HD), lambda i: (i, 0)),
            pl.BlockSpec((_BS1, NKV * HD), lambda i: (i, 0)),
            pl.BlockSpec((_BS1, NKV * HD), lambda i: (i, 0)),
        ],
        out_shape=[
            jax.ShapeDtypeStruct((S, NH * HD), f32),
            jax.ShapeDtypeStruct((S, NKV * HD), f32),
            jax.ShapeDtypeStruct((S, NKV * HD), f32),
        ],
    )(h, ln1_w.reshape(1, H), wq, wk, wv)

    ctx2 = pl.pallas_call(
        _attn_body,
        grid=(NH // _NHP, S // _BQ),
        in_specs=[
            pl.BlockSpec((_BQ, _NHP * HD), lambda g, qb: (qb, g)),
            pl.BlockSpec((S, 2 * HD), lambda g, qb: (0, g)),
            pl.BlockSpec((S, 2 * HD), lambda g, qb: (0, g)),
            pl.BlockSpec((_BQ, HD), lambda g, qb: (qb, 0)),
            pl.BlockSpec((_BQ, HD), lambda g, qb: (qb, 0)),
            pl.BlockSpec((S, HD), lambda g, qb: (0, 0)),
            pl.BlockSpec((S, HD), lambda g, qb: (0, 0)),
        ],
        out_specs=pl.BlockSpec((_BQ, _NHP * HD), lambda g, qb: (qb, g)),
        out_shape=jax.ShapeDtypeStruct((S, NH * HD), f32),
        scratch_shapes=[pltpu.VMEM((S, 2 * HD), f32)],
    )(q, k, v, cos64, sin64, cos64, sin64)

    nb3 = S // _BS3
    h1, x2, pw = pl.pallas_call(
        _out_router_body,
        grid=(nb3,),
        in_specs=[
            pl.BlockSpec((_BS3, NH * HD), lambda i: (i, 0)),
            pl.BlockSpec((_BS3, H), lambda i: (i, 0)),
            pl.BlockSpec((H, NH * HD), lambda i: (0, 0)),
            pl.BlockSpec((1, H), lambda i: (0, 0)),
            pl.BlockSpec((E, H), lambda i: (0, 0)),
            pl.BlockSpec((1, E), lambda i: (0, 0)),
        ],
        out_specs=[
            pl.BlockSpec((_BS3, H), lambda i: (i, 0)),
            pl.BlockSpec((_BS3, H), lambda i: (i, 0)),
            pl.BlockSpec((_BS3, E), lambda i: (i, 0)),
        ],
        out_shape=[
            jax.ShapeDtypeStruct((S, H), f32),
            jax.ShapeDtypeStruct((S, H), f32),
            jax.ShapeDtypeStruct((S, E), f32),
        ],
    )(ctx2, h, wo, ln2_w.reshape(1, H), gate_w, gate_b.reshape(1, E))

    out = pl.pallas_call(
        _moe_body,
        grid=(E, I // _BID),
        in_specs=[
            pl.BlockSpec((S, H), lambda e, i: (0, 0)),
            pl.BlockSpec((S, H), lambda e, i: (0, 0)),
            pl.BlockSpec((S, E), lambda e, i: (0, 0)),
            pl.BlockSpec((1, _BID, H), lambda e, i: (e, i, 0)),
            pl.BlockSpec((1, _BID, H), lambda e, i: (e, i, 0)),
            pl.BlockSpec((1, H, _BID), lambda e, i: (e, 0, i)),
        ],
        out_specs=pl.BlockSpec((S, H), lambda e, i: (0, 0)),
        out_shape=jax.ShapeDtypeStruct((S, H), f32),
    )(x2, h1, pw, w1, w3, w2)
    return out


# speedup vs baseline: 1.2753x; 1.0620x over previous
import functools
import math

import jax
import jax.numpy as jnp
import numpy as np
from jax import lax
from jax.experimental import pallas as pl
from jax.experimental.pallas import tpu as pltpu
from jax.experimental.pallas import tpu_sc as plsc

S = 2048
H = 1024
NH = 16
NKV = 8
HD = 64
I = 3584
E = 8
K = 2
THETA = 10000.0
EPS = 1e-06
NEG = float(jnp.finfo(jnp.float32).min)

_BS1 = 256
_BQ = 256
_BS3 = 512
_BI = 512
_GBS = 128
_BID = 512
_PMAX = S * K + E * _GBS
_NB = _PMAX // _GBS
_NW = 32


def _dot_t(a, b):
    return lax.dot_general(a, b, (((1,), (1,)), ((), ())),
                           preferred_element_type=jnp.float32)


def _qkv_body(h_ref, ln1_ref, wq_ref, wk_ref, wv_ref, q_ref, k_ref, v_ref):
    x = h_ref[...]
    var = jnp.mean(x * x, axis=1, keepdims=True)
    xn = (x * lax.rsqrt(var + EPS)) * ln1_ref[...]
    q_ref[...] = _dot_t(xn, wq_ref[...])
    k_ref[...] = _dot_t(xn, wk_ref[...])
    v_ref[...] = _dot_t(xn, wv_ref[...])


def _rope(x, c, s):
    half = x.shape[1] // 2
    xr = jnp.concatenate([-x[:, half:], x[:, :half]], axis=1)
    return x * c + xr * s


_NHP = 4


def _attn_body(q_ref, k_ref, v_ref, cq_ref, sq_ref, ck_ref, sk_ref,
               o_ref, kr_ref):
    qb = pl.program_id(1)
    c64, s64 = ck_ref[...], sk_ref[...]
    kr_ref[...] = jnp.concatenate(
        [_rope(k_ref[:, :HD], c64, s64), _rope(k_ref[:, HD:], c64, s64)],
        axis=1)
    scale = 1.0 / math.sqrt(HD)
    row = lax.broadcasted_iota(jnp.int32, (_BQ, _BQ), 0)
    col = lax.broadcasted_iota(jnp.int32, (_BQ, _BQ), 1)

    outs = []
    for hh in range(_NHP):
        q = _rope(q_ref[:, hh * HD:(hh + 1) * HD], cq_ref[...], sq_ref[...])
        kvc = (hh // 2) * HD

        def chunk(c, carry, q=q, kvc=kvc):
            m, l, acc = carry
            kc = kr_ref[pl.ds(c * _BQ, _BQ), kvc:kvc + HD]
            s = _dot_t(q, kc) * scale
            s = jnp.where((qb - c) * _BQ + row >= col, s, NEG)
            m2 = jnp.maximum(m, jnp.max(s, axis=1, keepdims=True))
            p = jnp.exp(s - m2)
            corr = jnp.exp(m - m2)
            l2 = l * corr + jnp.sum(p, axis=1, keepdims=True)
            vc = v_ref[pl.ds(c * _BQ, _BQ), kvc:kvc + HD]
            acc2 = acc * corr + jnp.dot(p, vc,
                                        preferred_element_type=jnp.float32)
            return m2, l2, acc2

        m0 = jnp.full((_BQ, 1), NEG, dtype=jnp.float32)
        l0 = jnp.zeros((_BQ, 1), dtype=jnp.float32)
        a0 = jnp.zeros((_BQ, HD), dtype=jnp.float32)
        m, l, acc = lax.fori_loop(0, qb + 1, chunk, (m0, l0, a0))
        outs.append(acc / l)
    o_ref[...] = jnp.concatenate(outs, axis=1)


def _out_router_body(ctx_ref, h_ref, wo_ref, ln2_ref, gw_ref, gb_ref,
                     h1_ref, x2_ref, pw_ref):
    attn_out = _dot_t(ctx_ref[...], wo_ref[...])
    h1 = h_ref[...] + attn_out
    h1_ref[...] = h1
    var = jnp.mean(h1 * h1, axis=1, keepdims=True)
    x2 = (h1 * lax.rsqrt(var + EPS)) * ln2_ref[...]
    x2_ref[...] = x2
    logits = _dot_t(x2, gw_ref[...]) + gb_ref[...]
    mx = jnp.max(logits, axis=1, keepdims=True)
    ex = jnp.exp(logits - mx)
    probs = ex / jnp.sum(ex, axis=1, keepdims=True)
    idx = lax.broadcasted_iota(jnp.int32, probs.shape, 1)
    m1 = jnp.max(probs, axis=1, keepdims=True)
    c1 = jnp.where(probs == m1, idx, E)
    i1 = jnp.min(c1, axis=1, keepdims=True)
    p2 = jnp.where(idx == i1, -1.0, probs)
    m2 = jnp.max(p2, axis=1, keepdims=True)
    c2 = jnp.where(p2 == m2, idx, E)
    i2 = jnp.min(c2, axis=1, keepdims=True)
    pw_ref[...] = jnp.where(idx == i1, m1, jnp.where(idx == i2, m2, 0.0))


def _moe_body(x2_ref, h1_ref, pw_ref, w1_ref, w3_ref, w2_ref, o_ref):
    e = pl.program_id(0)
    i = pl.program_id(1)
    x2 = x2_ref[...]
    a1 = _dot_t(x2, w1_ref[0])
    a3 = _dot_t(x2, w3_ref[0])
    g = (a1 / (1.0 + jnp.exp(-a1))) * a3
    part = _dot_t(g, w2_ref[0])
    sel = lax.broadcasted_iota(jnp.int32, (S, E), 1) == e
    col = jnp.sum(jnp.where(sel, pw_ref[...], 0.0), axis=1, keepdims=True)
    contrib = col * part

    @pl.when((e == 0) & (i == 0))
    def _init():
        o_ref[...] = h1_ref[...] + contrib

    @pl.when((e > 0) | (i > 0))
    def _acc():
        o_ref[...] += contrib


def kernel(h, ln1_w, ln2_w, wq, wk, wv, wo, gate_w, gate_b, w1, w2, w3):
    f32 = jnp.float32
    inv = 1.0 / (THETA ** (np.arange(0, HD, 2, dtype=np.float32) / HD))
    t = np.arange(S, dtype=np.float32)
    f_a = np.concatenate([np.outer(t, inv)] * 2, axis=1)
    cos64 = jnp.asarray(np.cos(f_a), dtype=f32)
    sin64 = jnp.asarray(np.sin(f_a), dtype=f32)

    nb1 = S // _BS1
    q, k, v = pl.pallas_call(
        _qkv_body,
        grid=(nb1,),
        in_specs=[
            pl.BlockSpec((_BS1, H), lambda i: (i, 0)),
            pl.BlockSpec((1, H), lambda i: (0, 0)),
            pl.BlockSpec((NH * HD, H), lambda i: (0, 0)),
            pl.BlockSpec((NKV * HD, H), lambda i: (0, 0)),
            pl.BlockSpec((NKV * HD, H), lambda i: (0, 0)),
        ],
        out_specs=[
            pl.BlockSpec((_BS1, NH * HD), lambda i: (i, 0)),
            pl.BlockSpec((_BS1, NKV * HD), lambda i: (i, 0)),
            pl.BlockSpec((_BS1, NKV * HD), lambda i: (i, 0)),
        ],
        out_shape=[
            jax.ShapeDtypeStruct((S, NH * HD), f32),
            jax.ShapeDtypeStruct((S, NKV * HD), f32),
            jax.ShapeDtypeStruct((S, NKV * HD), f32),
        ],
    )(h, ln1_w.reshape(1, H), wq, wk, wv)

    ctx2 = pl.pallas_call(
        _attn_body,
        grid=(NH // _NHP, S // _BQ),
        in_specs=[
            pl.BlockSpec((_BQ, _NHP * HD), lambda g, qb: (qb, g)),
            pl.BlockSpec((S, 2 * HD), lambda g, qb: (0, g)),
            pl.BlockSpec((S, 2 * HD), lambda g, qb: (0, g)),
            pl.BlockSpec((_BQ, HD), lambda g, qb: (qb, 0)),
            pl.BlockSpec((_BQ, HD), lambda g, qb: (qb, 0)),
            pl.BlockSpec((S, HD), lambda g, qb: (0, 0)),
            pl.BlockSpec((S, HD), lambda g, qb: (0, 0)),
        ],
        out_specs=pl.BlockSpec((_BQ, _NHP * HD), lambda g, qb: (qb, g)),
        out_shape=jax.ShapeDtypeStruct((S, NH * HD), f32),
        scratch_shapes=[pltpu.VMEM((S, 2 * HD), f32)],
    )(q, k, v, cos64, sin64, cos64, sin64)

    nb3 = S // _BS3
    h1, x2, pw = pl.pallas_call(
        _out_router_body,
        grid=(nb3,),
        in_specs=[
            pl.BlockSpec((_BS3, NH * HD), lambda i: (i, 0)),
            pl.BlockSpec((_BS3, H), lambda i: (i, 0)),
            pl.BlockSpec((H, NH * HD), lambda i: (0, 0)),
            pl.BlockSpec((1, H), lambda i: (0, 0)),
            pl.BlockSpec((E, H), lambda i: (0, 0)),
            pl.BlockSpec((1, E), lambda i: (0, 0)),
        ],
        out_specs=[
            pl.BlockSpec((_BS3, H), lambda i: (i, 0)),
            pl.BlockSpec((_BS3, H), lambda i: (i, 0)),
            pl.BlockSpec((_BS3, E), lambda i: (i, 0)),
        ],
        out_shape=[
            jax.ShapeDtypeStruct((S, H), f32),
            jax.ShapeDtypeStruct((S, H), f32),
            jax.ShapeDtypeStruct((S, E), f32),
        ],
    )(ctx2, h, wo, ln2_w.reshape(1, H), gate_w, gate_b.reshape(1, E))

    out = pl.pallas_call(
        _moe_body,
        grid=(E, I // _BID),
        in_specs=[
            pl.BlockSpec((S, H), lambda e, i: (0, 0)),
            pl.BlockSpec((S, H), lambda e, i: (0, 0)),
            pl.BlockSpec((S, E), lambda e, i: (0, 0)),
            pl.BlockSpec((1, _BID, H), lambda e, i: (e, i, 0)),
            pl.BlockSpec((1, _BID, H), lambda e, i: (e, i, 0)),
            pl.BlockSpec((1, H, _BID), lambda e, i: (e, 0, i)),
        ],
        out_specs=pl.BlockSpec((S, H), lambda e, i: (0, 0)),
        out_shape=jax.ShapeDtypeStruct((S, H), f32),
    )(x2, h1, pw, w1, w3, w2)
    return out
